# baseline (speedup 1.0000x reference)
import jax
import jax.numpy as jnp
from jax.experimental import pallas as pl
from jax.experimental.pallas import tpu as pltpu

_D = 256
_ROWS = 32 * 1024
_N_CHUNKS = 16
_CHUNK = _ROWS // _N_CHUNKS


def _copy_body(x_ref, o_ref, buf, in_sems, out_sems):
    for i in range(_N_CHUNKS):
        pltpu.make_async_copy(
            x_ref.at[pl.ds(i * _CHUNK, _CHUNK)], buf.at[i], in_sems.at[i]
        ).start()
    for i in range(_N_CHUNKS):
        pltpu.make_async_copy(
            x_ref.at[pl.ds(i * _CHUNK, _CHUNK)], buf.at[i], in_sems.at[i]
        ).wait()
        pltpu.make_async_copy(
            buf.at[i], o_ref.at[pl.ds(i * _CHUNK, _CHUNK)], out_sems.at[i]
        ).start()
    for i in range(_N_CHUNKS):
        pltpu.make_async_copy(
            buf.at[i], o_ref.at[pl.ds(i * _CHUNK, _CHUNK)], out_sems.at[i]
        ).wait()


def kernel(x):
    x2 = x.reshape(-1, _D)
    return pl.pallas_call(
        _copy_body,
        in_specs=[pl.BlockSpec(memory_space=pl.ANY)],
        out_specs=pl.BlockSpec(memory_space=pl.ANY),
        out_shape=jax.ShapeDtypeStruct((_ROWS, _D), x2.dtype),
        scratch_shapes=[
            pltpu.VMEM((_N_CHUNKS, _CHUNK, _D), jnp.float32),
            pltpu.SemaphoreType.DMA((_N_CHUNKS,)),
            pltpu.SemaphoreType.DMA((_N_CHUNKS,)),
        ],
    )(x2)
